# baseline (speedup 1.0000x reference)
import functools
import jax
import jax.numpy as jnp
from jax import lax
from jax.experimental import pallas as pl
from jax.experimental.pallas import tpu as pltpu
from jax.experimental.pallas import tpu_sc as plsc

_BLK = 128
_NC = 2
_NS = 16


def _silu(v):
    return v * jax.nn.sigmoid(v)


def _dot(a, b):
    return jnp.dot(a, b, preferred_element_type=jnp.float32)


def _pre_body(x_ref, wa_ref, wb_ref, a_ref, b_ref):
    x = x_ref[...]
    a_ref[...] = _dot(x, wa_ref[...])
    b_ref[...] = _dot(x, wb_ref[...])


def _edge_body(ea_ref, p_ref, we1_ref, be1_ref, we2_ref, be2_ref,
               w1c_ref, bm1_ref, wm2_ref, bm2_ref, wg_ref, bg_ref, m_ref):
    h = _silu(_dot(ea_ref[...], we1_ref[...]) + be1_ref[...])
    emb = _dot(h, we2_ref[...]) + be2_ref[...]
    t = _silu(p_ref[...] + _dot(emb, w1c_ref[...]) + bm1_ref[...])
    g = jax.nn.sigmoid(_dot(emb, wg_ref[...]) + bg_ref[...])
    m_ref[...] = (_dot(t, wm2_ref[...]) + bm2_ref[...]) * g


def _upd_body(p_ref, x_ref, wua_ref, wub_ref, bu1_ref, wu2_ref, bu2_ref,
              lng_ref, lnb_ref, out_ref):
    x = x_ref[...]
    n = x.shape[0]
    aggr = p_ref[0, :n] + p_ref[1, :n]
    u = _silu(_dot(aggr, wua_ref[...]) + _dot(x, wub_ref[...]) + bu1_ref[...])
    h = x + _dot(u, wu2_ref[...]) + bu2_ref[...]
    mu = jnp.mean(h, axis=-1, keepdims=True)
    d = h - mu
    var = jnp.mean(d * d, axis=-1, keepdims=True)
    out_ref[...] = d * lax.rsqrt(var + 1e-5) * lng_ref[...] + lnb_ref[...]


def _make_gather(E, D):
    nw = _NC * _NS
    epw = E // nw
    nblk = epw // _BLK
    mesh = plsc.VectorSubcoreMesh(core_axis_name="c", subcore_axis_name="s")

    @functools.partial(
        pl.kernel, mesh=mesh,
        out_type=jax.ShapeDtypeStruct((E, D), jnp.float32),
        scratch_types=[
            pltpu.VMEM((nblk, _BLK), jnp.int32),
            pltpu.VMEM((nblk, _BLK), jnp.int32),
            pltpu.VMEM((4, _BLK, D), jnp.float32),
        ] + [pltpu.SemaphoreType.DMA] * 12)
    def gather_k(a_hbm, b_hbm, dsti_hbm, srci_hbm, p_hbm,
                 di_v, si_v, r_v, *sems):
        sa = sems[0:4]
        sb = sems[4:8]
        ss = sems[8:12]
        c = lax.axis_index("c")
        s = lax.axis_index("s")
        wid = s * _NC + c
        pltpu.sync_copy(dsti_hbm.at[wid], di_v)
        pltpu.sync_copy(srci_hbm.at[wid], si_v)

        def body(q, carry):
            j0 = q * 4
            for k in range(4):
                pltpu.async_copy(a_hbm.at[di_v.at[j0 + k]], r_v.at[k],
                                 sa[k]).wait()
                pltpu.async_copy(b_hbm.at[si_v.at[j0 + k]],
                                 r_v.at[k], sb[k], add=True).wait()
                pltpu.async_copy(
                    r_v.at[k],
                    p_hbm.at[pl.ds(wid * epw + (j0 + k) * _BLK, _BLK)],
                    ss[k]).wait()
            return carry

        lax.fori_loop(0, nblk // 4, body, 0)

    return gather_k


def _make_scatter(E, N, D, npad):
    blk = 80
    epc = E // _NC
    ept = epc // _NS
    nblk = ept // blk
    rpt = npad // _NS
    zrows = 80
    mesh = plsc.VectorSubcoreMesh(core_axis_name="c", subcore_axis_name="s")

    @functools.partial(
        pl.kernel, mesh=mesh,
        out_type=jax.ShapeDtypeStruct((_NC, npad, D), jnp.float32),
        scratch_types=[
            pltpu.VMEM((nblk, blk), jnp.int32),
            pltpu.VMEM((blk, D), jnp.float32),
            pltpu.VMEM((zrows, D), jnp.float32),
            pltpu.VMEM_SHARED((npad, D), jnp.float32),
        ])
    def scatter_k(msg_hbm, dsti_hbm, out_hbm, di_v, m_v, z_v, acc_sh):
        c = lax.axis_index("c")
        s = lax.axis_index("s")

        def zero_body(i, carry):
            for jj in range(D // 16):
                z_v[i, pl.ds(jj * 16, 16)] = jnp.zeros((16,), jnp.float32)
            return carry

        lax.fori_loop(0, zrows, zero_body, 0)
        for k in range(rpt // zrows):
            pltpu.sync_copy(z_v, acc_sh.at[pl.ds(s * rpt + k * zrows, zrows)])
        plsc.subcore_barrier()

        base_e = c * epc + s * ept
        pltpu.sync_copy(dsti_hbm.at[c * _NS + s], di_v)

        def body(j, carry):
            pltpu.sync_copy(msg_hbm.at[pl.ds(base_e + j * blk, blk)], m_v)
            pltpu.sync_copy(m_v, acc_sh.at[di_v.at[j]], add=True)
            return carry

        lax.fori_loop(0, nblk, body, 0)
        plsc.subcore_barrier()
        pltpu.sync_copy(acc_sh.at[pl.ds(s * rpt, rpt)],
                        out_hbm.at[c, pl.ds(s * rpt, rpt)])

    return scatter_k


def kernel(x, edge_index, edge_attr, We1, be1, We2, be2, Wm1, bm1, Wm2, bm2,
           Wu1, bu1, Wu2, bu2, Wg, bg, ln_gamma, ln_beta):
    N, D = x.shape
    E, R = edge_attr.shape
    f32 = jnp.float32

    nw = _NC * _NS
    npad = ((N + nw * 8 - 1) // (nw * 8)) * nw * 8
    ep = ((E + nw * 640 - 1) // (nw * 640)) * nw * 640
    pad_idx = (N + jnp.arange(ep - E, dtype=jnp.int32) % (npad - N))
    src2 = jnp.concatenate([edge_index[0], pad_idx]).reshape(nw, -1, _BLK)
    dst2 = jnp.concatenate([edge_index[1], pad_idx]).reshape(nw, -1, _BLK)
    ea_pad = jnp.zeros((ep, R), jnp.float32).at[:E].set(edge_attr)
    x_pad = jnp.zeros((npad, D), jnp.float32).at[:N].set(x)
    W1a, W1b, W1c = Wm1[:D], Wm1[D:2 * D], Wm1[2 * D:]
    Wua, Wub = Wu1[:D], Wu1[D:]
    row = lambda v: v.reshape(1, -1)

    A, B = pl.pallas_call(
        _pre_body,
        out_shape=(jax.ShapeDtypeStruct((npad, D), f32),
                   jax.ShapeDtypeStruct((npad, D), f32)),
    )(x_pad, W1a, W1b)

    P = _make_gather(ep, D)(A, B, dst2, src2)

    EB = 8192
    full = lambda w: pl.BlockSpec(w.shape, lambda i: (0,) * w.ndim)
    msg = pl.pallas_call(
        _edge_body,
        grid=(ep // EB,),
        in_specs=[
            pl.BlockSpec((EB, R), lambda i: (i, 0)),
            pl.BlockSpec((EB, D), lambda i: (i, 0)),
            full(We1), full(row(be1)), full(We2), full(row(be2)),
            full(W1c), full(row(bm1)), full(Wm2), full(row(bm2)),
            full(Wg), full(row(bg)),
        ],
        out_specs=pl.BlockSpec((EB, D), lambda i: (i, 0)),
        out_shape=jax.ShapeDtypeStruct((ep, D), f32),
    )(ea_pad, P, We1, row(be1), We2, row(be2), W1c, row(bm1),
      Wm2, row(bm2), Wg, row(bg))

    dst2s = dst2.reshape(_NC * _NS, -1, 80)
    partials = _make_scatter(ep, N, D, npad)(msg, dst2s)

    out = pl.pallas_call(
        _upd_body,
        out_shape=jax.ShapeDtypeStruct((N, D), f32),
    )(partials, x, Wua, Wub, row(bu1), Wu2, row(bu2),
      row(ln_gamma), row(ln_beta))
    return out


# speedup vs baseline: 1.0933x; 1.0933x over previous
import functools
import jax
import jax.numpy as jnp
from jax import lax
from jax.experimental import pallas as pl
from jax.experimental.pallas import tpu as pltpu
from jax.experimental.pallas import tpu_sc as plsc

_BLK = 128
_NC = 2
_NS = 16


def _silu(v):
    return v * jax.nn.sigmoid(v)


def _dot(a, b):
    return jnp.dot(a, b, preferred_element_type=jnp.float32)


def _pre_body(x_ref, wa_ref, wb_ref, a_ref, b_ref):
    x = x_ref[...]
    a_ref[...] = _dot(x, wa_ref[...])
    b_ref[...] = _dot(x, wb_ref[...])


def _edge_body(ea_ref, p_ref, we1_ref, be1_ref, we2_ref, be2_ref,
               w1c_ref, bm1_ref, wm2_ref, bm2_ref, wg_ref, bg_ref, m_ref):
    h = _silu(_dot(ea_ref[...], we1_ref[...]) + be1_ref[...])
    emb = _dot(h, we2_ref[...]) + be2_ref[...]
    t = _silu(p_ref[...] + _dot(emb, w1c_ref[...]) + bm1_ref[...])
    g = jax.nn.sigmoid(_dot(emb, wg_ref[...]) + bg_ref[...])
    m_ref[...] = (_dot(t, wm2_ref[...]) + bm2_ref[...]) * g


def _upd_body(p_ref, x_ref, wua_ref, wub_ref, bu1_ref, wu2_ref, bu2_ref,
              lng_ref, lnb_ref, out_ref):
    x = x_ref[...]
    n = x.shape[0]
    aggr = p_ref[0, :n] + p_ref[1, :n]
    u = _silu(_dot(aggr, wua_ref[...]) + _dot(x, wub_ref[...]) + bu1_ref[...])
    h = x + _dot(u, wu2_ref[...]) + bu2_ref[...]
    mu = jnp.mean(h, axis=-1, keepdims=True)
    d = h - mu
    var = jnp.mean(d * d, axis=-1, keepdims=True)
    out_ref[...] = d * lax.rsqrt(var + 1e-5) * lng_ref[...] + lnb_ref[...]


def _make_gather(E, D):
    nw = _NC * _NS
    epw = E // nw
    nblk = epw // _BLK
    mesh = plsc.VectorSubcoreMesh(core_axis_name="c", subcore_axis_name="s")

    @functools.partial(
        pl.kernel, mesh=mesh,
        out_type=jax.ShapeDtypeStruct((E, D), jnp.float32),
        scratch_types=[
            pltpu.VMEM((nblk, _BLK), jnp.int32),
            pltpu.VMEM((nblk, _BLK), jnp.int32),
            pltpu.VMEM((4, _BLK, D), jnp.float32),
        ] + [pltpu.SemaphoreType.DMA] * 12)
    def gather_k(a_hbm, b_hbm, dsti_hbm, srci_hbm, p_hbm,
                 di_v, si_v, r_v, *sems):
        sa = sems[0:4]
        sb = sems[4:8]
        ss = sems[8:12]
        c = lax.axis_index("c")
        s = lax.axis_index("s")
        wid = s * _NC + c
        pltpu.sync_copy(dsti_hbm.at[wid], di_v)
        pltpu.sync_copy(srci_hbm.at[wid], si_v)

        def body(q, carry):
            j0 = q * 4
            ga = [pltpu.async_copy(a_hbm.at[di_v.at[j0 + k]], r_v.at[k],
                                   sa[k])
                  for k in range(4)]
            gb = []
            for k in range(4):
                ga[k].wait()
                gb.append(pltpu.async_copy(b_hbm.at[si_v.at[j0 + k]],
                                           r_v.at[k], sb[k], add=True))
            st = []
            for k in range(4):
                gb[k].wait()
                st.append(pltpu.async_copy(
                    r_v.at[k],
                    p_hbm.at[pl.ds(wid * epw + (j0 + k) * _BLK, _BLK)],
                    ss[k]))
            for k in range(4):
                st[k].wait()
            return carry

        lax.fori_loop(0, nblk // 4, body, 0)

    return gather_k


def _make_scatter(E, N, D, npad):
    blk = 80
    epc = E // _NC
    ept = epc // _NS
    nblk = ept // blk
    rpt = npad // _NS
    zrows = 80
    mesh = plsc.VectorSubcoreMesh(core_axis_name="c", subcore_axis_name="s")

    @functools.partial(
        pl.kernel, mesh=mesh,
        out_type=jax.ShapeDtypeStruct((_NC, npad, D), jnp.float32),
        scratch_types=[
            pltpu.VMEM((nblk, blk), jnp.int32),
            pltpu.VMEM((blk, D), jnp.float32),
            pltpu.VMEM((zrows, D), jnp.float32),
            pltpu.VMEM_SHARED((npad, D), jnp.float32),
        ])
    def scatter_k(msg_hbm, dsti_hbm, out_hbm, di_v, m_v, z_v, acc_sh):
        c = lax.axis_index("c")
        s = lax.axis_index("s")

        def zero_body(i, carry):
            for jj in range(D // 16):
                z_v[i, pl.ds(jj * 16, 16)] = jnp.zeros((16,), jnp.float32)
            return carry

        lax.fori_loop(0, zrows, zero_body, 0)
        for k in range(rpt // zrows):
            pltpu.sync_copy(z_v, acc_sh.at[pl.ds(s * rpt + k * zrows, zrows)])
        plsc.subcore_barrier()

        base_e = c * epc + s * ept
        pltpu.sync_copy(dsti_hbm.at[c * _NS + s], di_v)

        def body(j, carry):
            pltpu.sync_copy(msg_hbm.at[pl.ds(base_e + j * blk, blk)], m_v)
            pltpu.sync_copy(m_v, acc_sh.at[di_v.at[j]], add=True)
            return carry

        lax.fori_loop(0, nblk, body, 0)
        plsc.subcore_barrier()
        pltpu.sync_copy(acc_sh.at[pl.ds(s * rpt, rpt)],
                        out_hbm.at[c, pl.ds(s * rpt, rpt)])

    return scatter_k


def kernel(x, edge_index, edge_attr, We1, be1, We2, be2, Wm1, bm1, Wm2, bm2,
           Wu1, bu1, Wu2, bu2, Wg, bg, ln_gamma, ln_beta):
    N, D = x.shape
    E, R = edge_attr.shape
    f32 = jnp.float32

    nw = _NC * _NS
    npad = ((N + nw * 8 - 1) // (nw * 8)) * nw * 8
    ep = ((E + nw * 640 - 1) // (nw * 640)) * nw * 640
    pad_idx = (N + jnp.arange(ep - E, dtype=jnp.int32) % (npad - N))
    src2 = jnp.concatenate([edge_index[0], pad_idx]).reshape(nw, -1, _BLK)
    dst2 = jnp.concatenate([edge_index[1], pad_idx]).reshape(nw, -1, _BLK)
    ea_pad = jnp.zeros((ep, R), jnp.float32).at[:E].set(edge_attr)
    x_pad = jnp.zeros((npad, D), jnp.float32).at[:N].set(x)
    W1a, W1b, W1c = Wm1[:D], Wm1[D:2 * D], Wm1[2 * D:]
    Wua, Wub = Wu1[:D], Wu1[D:]
    row = lambda v: v.reshape(1, -1)

    A, B = pl.pallas_call(
        _pre_body,
        out_shape=(jax.ShapeDtypeStruct((npad, D), f32),
                   jax.ShapeDtypeStruct((npad, D), f32)),
    )(x_pad, W1a, W1b)

    P = _make_gather(ep, D)(A, B, dst2, src2)

    EB = 8192
    full = lambda w: pl.BlockSpec(w.shape, lambda i: (0,) * w.ndim)
    msg = pl.pallas_call(
        _edge_body,
        grid=(ep // EB,),
        in_specs=[
            pl.BlockSpec((EB, R), lambda i: (i, 0)),
            pl.BlockSpec((EB, D), lambda i: (i, 0)),
            full(We1), full(row(be1)), full(We2), full(row(be2)),
            full(W1c), full(row(bm1)), full(Wm2), full(row(bm2)),
            full(Wg), full(row(bg)),
        ],
        out_specs=pl.BlockSpec((EB, D), lambda i: (i, 0)),
        out_shape=jax.ShapeDtypeStruct((ep, D), f32),
    )(ea_pad, P, We1, row(be1), We2, row(be2), W1c, row(bm1),
      Wm2, row(bm2), Wg, row(bg))

    dst2s = dst2.reshape(_NC * _NS, -1, 80)
    partials = _make_scatter(ep, N, D, npad)(msg, dst2s)

    out = pl.pallas_call(
        _upd_body,
        out_shape=jax.ShapeDtypeStruct((N, D), f32),
    )(partials, x, Wua, Wub, row(bu1), Wu2, row(bu2),
      row(ln_gamma), row(ln_beta))
    return out


# speedup vs baseline: 1.1703x; 1.0704x over previous
import functools
import jax
import jax.numpy as jnp
from jax import lax
from jax.experimental import pallas as pl
from jax.experimental.pallas import tpu as pltpu
from jax.experimental.pallas import tpu_sc as plsc

_BLK = 128
_NC = 2
_NS = 16


def _silu(v):
    return v * jax.nn.sigmoid(v)


def _dot(a, b):
    return jnp.dot(a, b, preferred_element_type=jnp.float32)


def _pre_body(x_ref, wa_ref, wb_ref, a_ref, b_ref):
    x = x_ref[...]
    a_ref[...] = _dot(x, wa_ref[...])
    b_ref[...] = _dot(x, wb_ref[...])


def _edge_body(ea_ref, p_ref, we1_ref, be1_ref, we2_ref, be2_ref,
               w1c_ref, bm1_ref, wm2_ref, bm2_ref, wg_ref, bg_ref, m_ref):
    h = _silu(_dot(ea_ref[...], we1_ref[...]) + be1_ref[...])
    emb = _dot(h, we2_ref[...]) + be2_ref[...]
    t = _silu(p_ref[...] + _dot(emb, w1c_ref[...]) + bm1_ref[...])
    g = jax.nn.sigmoid(_dot(emb, wg_ref[...]) + bg_ref[...])
    m_ref[...] = (_dot(t, wm2_ref[...]) + bm2_ref[...]) * g


def _upd_body(p_ref, x_ref, wua_ref, wub_ref, bu1_ref, wu2_ref, bu2_ref,
              lng_ref, lnb_ref, out_ref):
    x = x_ref[...]
    n = x.shape[0]
    aggr = p_ref[0, :n] + p_ref[1, :n]
    u = _silu(_dot(aggr, wua_ref[...]) + _dot(x, wub_ref[...]) + bu1_ref[...])
    h = x + _dot(u, wu2_ref[...]) + bu2_ref[...]
    mu = jnp.mean(h, axis=-1, keepdims=True)
    d = h - mu
    var = jnp.mean(d * d, axis=-1, keepdims=True)
    out_ref[...] = d * lax.rsqrt(var + 1e-5) * lng_ref[...] + lnb_ref[...]


def _make_gather(E, D):
    nw = _NC * _NS
    epw = E // nw
    nblk = epw // _BLK
    mesh = plsc.VectorSubcoreMesh(core_axis_name="c", subcore_axis_name="s")

    @functools.partial(
        pl.kernel, mesh=mesh,
        out_type=jax.ShapeDtypeStruct((E, D), jnp.float32),
        scratch_types=[
            pltpu.VMEM((nblk, _BLK), jnp.int32),
            pltpu.VMEM((nblk, _BLK), jnp.int32),
            pltpu.VMEM((4, _BLK, D), jnp.float32),
        ] + [pltpu.SemaphoreType.DMA] * 12)
    def gather_k(a_hbm, b_hbm, dsti_hbm, srci_hbm, p_hbm,
                 di_v, si_v, r_v, *sems):
        sa = sems[0:4]
        sb = sems[4:8]
        ss = sems[8:12]
        c = lax.axis_index("c")
        s = lax.axis_index("s")
        wid = s * _NC + c
        pltpu.sync_copy(dsti_hbm.at[wid], di_v)
        pltpu.sync_copy(srci_hbm.at[wid], si_v)

        def body(q, carry):
            j0 = q * 4
            ga = [pltpu.async_copy(a_hbm.at[di_v.at[j0 + k]], r_v.at[k],
                                   sa[k])
                  for k in range(4)]
            gb = []
            for k in range(4):
                ga[k].wait()
                gb.append(pltpu.async_copy(b_hbm.at[si_v.at[j0 + k]],
                                           r_v.at[k], sb[k], add=True))
            st = []
            for k in range(4):
                gb[k].wait()
                st.append(pltpu.async_copy(
                    r_v.at[k],
                    p_hbm.at[pl.ds(wid * epw + (j0 + k) * _BLK, _BLK)],
                    ss[k]))
            for k in range(4):
                st[k].wait()
            return carry

        lax.fori_loop(0, nblk // 4, body, 0)

    return gather_k


def _make_scatter(E, N, D, npad):
    blk = 80
    epc = E // _NC
    ept = epc // _NS
    nblk = ept // blk
    rpt = npad // _NS
    zrows = 80
    mesh = plsc.VectorSubcoreMesh(core_axis_name="c", subcore_axis_name="s")

    @functools.partial(
        pl.kernel, mesh=mesh,
        out_type=jax.ShapeDtypeStruct((_NC, npad, D), jnp.float32),
        scratch_types=[
            pltpu.VMEM((nblk, blk), jnp.int32),
            pltpu.VMEM((2, blk, D), jnp.float32),
            pltpu.VMEM_SHARED((npad, D), jnp.float32),
        ] + [pltpu.SemaphoreType.DMA] * 4)
    def scatter_k(msg_hbm, dsti_hbm, out_hbm, di_v, m_v, acc_sh, *sems):
        sr = sems[0:2]
        sw = sems[2:4]
        c = lax.axis_index("c")
        s = lax.axis_index("s")

        def zero_body(i, carry):
            for jj in range(D // 16):
                m_v[0, i, pl.ds(jj * 16, 16)] = jnp.zeros((16,), jnp.float32)
            return carry

        lax.fori_loop(0, zrows, zero_body, 0)
        for k in range(rpt // zrows):
            pltpu.sync_copy(m_v.at[0],
                            acc_sh.at[pl.ds(s * rpt + k * zrows, zrows)])
        plsc.subcore_barrier()

        base_e = c * epc + s * ept
        pltpu.sync_copy(dsti_hbm.at[c * _NS + s], di_v)

        def body(q, carry):
            j0 = q * 2
            rd = [pltpu.async_copy(
                msg_hbm.at[pl.ds(base_e + (j0 + k) * blk, blk)],
                m_v.at[k], sr[k]) for k in range(2)]
            sc = []
            for k in range(2):
                rd[k].wait()
                sc.append(pltpu.async_copy(
                    m_v.at[k], acc_sh.at[di_v.at[j0 + k]], sw[k], add=True))
            for k in range(2):
                sc[k].wait()
            return carry

        lax.fori_loop(0, nblk // 2, body, 0)
        plsc.subcore_barrier()
        pltpu.sync_copy(acc_sh.at[pl.ds(s * rpt, rpt)],
                        out_hbm.at[c, pl.ds(s * rpt, rpt)])

    return scatter_k


def kernel(x, edge_index, edge_attr, We1, be1, We2, be2, Wm1, bm1, Wm2, bm2,
           Wu1, bu1, Wu2, bu2, Wg, bg, ln_gamma, ln_beta):
    N, D = x.shape
    E, R = edge_attr.shape
    f32 = jnp.float32

    nw = _NC * _NS
    npad = ((N + nw * 8 - 1) // (nw * 8)) * nw * 8
    ep = ((E + nw * 640 - 1) // (nw * 640)) * nw * 640
    pad_idx = (N + jnp.arange(ep - E, dtype=jnp.int32) % (npad - N))
    src2 = jnp.concatenate([edge_index[0], pad_idx]).reshape(nw, -1, _BLK)
    dst2 = jnp.concatenate([edge_index[1], pad_idx]).reshape(nw, -1, _BLK)
    ea_pad = jnp.zeros((ep, R), jnp.float32).at[:E].set(edge_attr)
    x_pad = jnp.zeros((npad, D), jnp.float32).at[:N].set(x)
    W1a, W1b, W1c = Wm1[:D], Wm1[D:2 * D], Wm1[2 * D:]
    Wua, Wub = Wu1[:D], Wu1[D:]
    row = lambda v: v.reshape(1, -1)

    A, B = pl.pallas_call(
        _pre_body,
        out_shape=(jax.ShapeDtypeStruct((npad, D), f32),
                   jax.ShapeDtypeStruct((npad, D), f32)),
    )(x_pad, W1a, W1b)

    P = _make_gather(ep, D)(A, B, dst2, src2)

    EB = 8192
    full = lambda w: pl.BlockSpec(w.shape, lambda i: (0,) * w.ndim)
    msg = pl.pallas_call(
        _edge_body,
        grid=(ep // EB,),
        in_specs=[
            pl.BlockSpec((EB, R), lambda i: (i, 0)),
            pl.BlockSpec((EB, D), lambda i: (i, 0)),
            full(We1), full(row(be1)), full(We2), full(row(be2)),
            full(W1c), full(row(bm1)), full(Wm2), full(row(bm2)),
            full(Wg), full(row(bg)),
        ],
        out_specs=pl.BlockSpec((EB, D), lambda i: (i, 0)),
        out_shape=jax.ShapeDtypeStruct((ep, D), f32),
    )(ea_pad, P, We1, row(be1), We2, row(be2), W1c, row(bm1),
      Wm2, row(bm2), Wg, row(bg))

    dst2s = dst2.reshape(_NC * _NS, -1, 80)
    partials = _make_scatter(ep, N, D, npad)(msg, dst2s)

    out = pl.pallas_call(
        _upd_body,
        out_shape=jax.ShapeDtypeStruct((N, D), f32),
    )(partials, x, Wua, Wub, row(bu1), Wu2, row(bu2),
      row(ln_gamma), row(ln_beta))
    return out


# speedup vs baseline: 1.2742x; 1.0888x over previous
import functools
import jax
import jax.numpy as jnp
from jax import lax
from jax.experimental import pallas as pl
from jax.experimental.pallas import tpu as pltpu
from jax.experimental.pallas import tpu_sc as plsc

_BLK = 128
_NC = 2
_NS = 16


def _silu(v):
    return v * jax.nn.sigmoid(v)


def _dot(a, b):
    return jnp.dot(a, b, preferred_element_type=jnp.float32)


def _pre_body(x_ref, wa_ref, wb_ref, a_ref, b_ref):
    x = x_ref[...]
    a_ref[...] = _dot(x, wa_ref[...])
    b_ref[...] = _dot(x, wb_ref[...])


def _edge_body(ea_ref, p_ref, we1_ref, be1_ref, we2_ref, be2_ref,
               w1c_ref, bm1_ref, wm2_ref, bm2_ref, wg_ref, bg_ref, m_ref):
    h = _silu(_dot(ea_ref[...], we1_ref[...]) + be1_ref[...])
    emb = _dot(h, we2_ref[...]) + be2_ref[...]
    t = _silu(p_ref[...] + _dot(emb, w1c_ref[...]) + bm1_ref[...])
    g = jax.nn.sigmoid(_dot(emb, wg_ref[...]) + bg_ref[...])
    m_ref[...] = (_dot(t, wm2_ref[...]) + bm2_ref[...]) * g


def _upd_body(p_ref, q_ref, x_ref, wua_ref, wub_ref, bu1_ref, wu2_ref,
              bu2_ref, lng_ref, lnb_ref, out_ref):
    x = x_ref[...]
    n = x.shape[0]
    aggr = (p_ref[0, :n] + p_ref[1, :n]) + (q_ref[0, :n] + q_ref[1, :n])
    u = _silu(_dot(aggr, wua_ref[...]) + _dot(x, wub_ref[...]) + bu1_ref[...])
    h = x + _dot(u, wu2_ref[...]) + bu2_ref[...]
    mu = jnp.mean(h, axis=-1, keepdims=True)
    d = h - mu
    var = jnp.mean(d * d, axis=-1, keepdims=True)
    out_ref[...] = d * lax.rsqrt(var + 1e-5) * lng_ref[...] + lnb_ref[...]


def _make_gather(E, D):
    nw = _NC * _NS
    epw = E // nw
    nblk = epw // _BLK
    mesh = plsc.VectorSubcoreMesh(core_axis_name="c", subcore_axis_name="s")

    @functools.partial(
        pl.kernel, mesh=mesh,
        out_type=jax.ShapeDtypeStruct((E, D), jnp.float32),
        scratch_types=[
            pltpu.VMEM((nblk, _BLK), jnp.int32),
            pltpu.VMEM((nblk, _BLK), jnp.int32),
            pltpu.VMEM((4, _BLK, D), jnp.float32),
        ] + [pltpu.SemaphoreType.DMA] * 12)
    def gather_k(a_hbm, b_hbm, dsti_hbm, srci_hbm, p_hbm,
                 di_v, si_v, r_v, *sems):
        sa = sems[0:4]
        sb = sems[4:8]
        ss = sems[8:12]
        c = lax.axis_index("c")
        s = lax.axis_index("s")
        wid = s * _NC + c
        pltpu.sync_copy(dsti_hbm.at[wid], di_v)
        pltpu.sync_copy(srci_hbm.at[wid], si_v)

        def body(q, carry):
            j0 = q * 4
            ga = [pltpu.async_copy(a_hbm.at[di_v.at[j0 + k]], r_v.at[k],
                                   sa[k])
                  for k in range(4)]
            gb = []
            for k in range(4):
                ga[k].wait()
                gb.append(pltpu.async_copy(b_hbm.at[si_v.at[j0 + k]],
                                           r_v.at[k], sb[k], add=True))
            st = []
            for k in range(4):
                gb[k].wait()
                st.append(pltpu.async_copy(
                    r_v.at[k],
                    p_hbm.at[pl.ds(wid * epw + (j0 + k) * _BLK, _BLK)],
                    ss[k]))
            for k in range(4):
                st[k].wait()
            return carry

        lax.fori_loop(0, nblk // 4, body, 0)

    return gather_k


def _make_scatter(E, N, D, npad):
    blk = 80
    epc = E // _NC
    ept = epc // _NS
    nblk = ept // blk
    rpt = npad // _NS
    zrows = 80
    mesh = plsc.VectorSubcoreMesh(core_axis_name="c", subcore_axis_name="s")

    @functools.partial(
        pl.kernel, mesh=mesh,
        out_type=jax.ShapeDtypeStruct((_NC, npad, D), jnp.float32),
        scratch_types=[
            pltpu.VMEM((nblk, blk), jnp.int32),
            pltpu.VMEM((2, blk, D), jnp.float32),
            pltpu.VMEM_SHARED((npad, D), jnp.float32),
        ] + [pltpu.SemaphoreType.DMA] * 4)
    def scatter_k(msg_hbm, dsti_hbm, out_hbm, di_v, m_v, acc_sh, *sems):
        sr = sems[0:2]
        sw = sems[2:4]
        c = lax.axis_index("c")
        s = lax.axis_index("s")

        def zero_body(i, carry):
            for jj in range(D // 16):
                m_v[0, i, pl.ds(jj * 16, 16)] = jnp.zeros((16,), jnp.float32)
            return carry

        lax.fori_loop(0, zrows, zero_body, 0)
        for k in range(rpt // zrows):
            pltpu.sync_copy(m_v.at[0],
                            acc_sh.at[pl.ds(s * rpt + k * zrows, zrows)])
        plsc.subcore_barrier()

        base_e = c * epc + s * ept
        pltpu.sync_copy(dsti_hbm.at[c * _NS + s], di_v)

        def body(q, carry):
            j0 = q * 2
            rd = [pltpu.async_copy(
                msg_hbm.at[pl.ds(base_e + (j0 + k) * blk, blk)],
                m_v.at[k], sr[k]) for k in range(2)]
            sc = []
            for k in range(2):
                rd[k].wait()
                sc.append(pltpu.async_copy(
                    m_v.at[k], acc_sh.at[di_v.at[j0 + k]], sw[k], add=True))
            for k in range(2):
                sc[k].wait()
            return carry

        lax.fori_loop(0, nblk // 2, body, 0)
        plsc.subcore_barrier()
        pltpu.sync_copy(acc_sh.at[pl.ds(s * rpt, rpt)],
                        out_hbm.at[c, pl.ds(s * rpt, rpt)])

    return scatter_k


def kernel(x, edge_index, edge_attr, We1, be1, We2, be2, Wm1, bm1, Wm2, bm2,
           Wu1, bu1, Wu2, bu2, Wg, bg, ln_gamma, ln_beta):
    N, D = x.shape
    E, R = edge_attr.shape
    f32 = jnp.float32

    nw = _NC * _NS
    npad = ((N + nw * 8 - 1) // (nw * 8)) * nw * 8
    ep = ((E + nw * 640 - 1) // (nw * 640)) * nw * 640
    pad_idx = (N + jnp.arange(ep - E, dtype=jnp.int32) % (npad - N))
    eh = ep // 2
    src_f = jnp.concatenate([edge_index[0], pad_idx])
    dst_f = jnp.concatenate([edge_index[1], pad_idx])
    src2 = [src_f[h * eh:(h + 1) * eh].reshape(nw, -1, _BLK) for h in (0, 1)]
    dst2 = [dst_f[h * eh:(h + 1) * eh].reshape(nw, -1, _BLK) for h in (0, 1)]
    ea_pad = jnp.zeros((ep, R), jnp.float32).at[:E].set(edge_attr)
    x_pad = jnp.zeros((npad, D), jnp.float32).at[:N].set(x)
    W1a, W1b, W1c = Wm1[:D], Wm1[D:2 * D], Wm1[2 * D:]
    Wua, Wub = Wu1[:D], Wu1[D:]
    row = lambda v: v.reshape(1, -1)

    A, B = pl.pallas_call(
        _pre_body,
        out_shape=(jax.ShapeDtypeStruct((npad, D), f32),
                   jax.ShapeDtypeStruct((npad, D), f32)),
    )(x_pad, W1a, W1b)

    EB = 8192
    full = lambda w: pl.BlockSpec(w.shape, lambda i: (0,) * w.ndim)
    gather_h = _make_gather(eh, D)
    scatter_h = _make_scatter(eh, N, D, npad)
    edge_h = pl.pallas_call(
        _edge_body,
        grid=(eh // EB,),
        in_specs=[
            pl.BlockSpec((EB, R), lambda i: (i, 0)),
            pl.BlockSpec((EB, D), lambda i: (i, 0)),
            full(We1), full(row(be1)), full(We2), full(row(be2)),
            full(W1c), full(row(bm1)), full(Wm2), full(row(bm2)),
            full(Wg), full(row(bg)),
        ],
        out_specs=pl.BlockSpec((EB, D), lambda i: (i, 0)),
        out_shape=jax.ShapeDtypeStruct((eh, D), f32),
    )

    partials = []
    for h in (0, 1):
        P = gather_h(A, B, dst2[h], src2[h])
        msg = edge_h(ea_pad[h * eh:(h + 1) * eh], P, We1, row(be1), We2,
                     row(be2), W1c, row(bm1), Wm2, row(bm2), Wg, row(bg))
        dst2s = dst2[h].reshape(_NC * _NS, -1, 80)
        partials.append(scatter_h(msg, dst2s))

    out = pl.pallas_call(
        _upd_body,
        out_shape=jax.ShapeDtypeStruct((N, D), f32),
    )(partials[0], partials[1], x, Wua, Wub, row(bu1), Wu2, row(bu2),
      row(ln_gamma), row(ln_beta))
    return out
